# baseline (speedup 1.0000x reference)
import jax
import jax.numpy as jnp
from jax.experimental import pallas as pl
from jax.experimental.pallas import tpu as pltpu

ROWS = 100000
COL = 128
LAT = 16
K = 64
B = 25000
NB = ROWS // B


def _dot(a, b, dims):
    return jax.lax.dot_general(a, b, (dims, ((), ())),
                               preferred_element_type=jnp.float32)


def _fused_kernel(tab_ref, w1t_ref, b1_ref, w2t_ref, b2_ref, wm_ref, bm_ref,
                  out_ref,
                  z_scr, lab_scr, dist_scr, cent0_scr, c2_scr, cent_scr,
                  sums_scr, cnt_scr, mm_scr):
    step = pl.program_id(0)

    @pl.when(step < NB)
    def _phase_a():
        b = step
        x = tab_ref[0]
        h1 = jnp.maximum(_dot(x, w1t_ref[...], ((1,), (0,))) + b1_ref[...], 0.0)
        h2 = jnp.maximum(_dot(h1, w2t_ref[...], ((1,), (0,))) + b2_ref[...], 0.0)
        zbt = _dot(wm_ref[...], h2, ((1,), (1,))) + bm_ref[...]
        z_scr[b] = zbt

        @pl.when(b == 0)
        def _():
            c0 = zbt[:, :K]
            cent0_scr[...] = c0
            c2row = jnp.sum(c0 * c0, axis=0, keepdims=True)
            eye = (jax.lax.broadcasted_iota(jnp.int32, (K, K), 0)
                   == jax.lax.broadcasted_iota(jnp.int32, (K, K), 1))
            c2_scr[...] = jnp.sum(
                jnp.where(eye, jnp.broadcast_to(c2row, (K, K)), 0.0),
                axis=1, keepdims=True)
            sums_scr[...] = jnp.zeros_like(sums_scr)
            cnt_scr[...] = jnp.zeros_like(cnt_scr)

        m2 = _dot(-2.0 * cent0_scr[...], zbt, ((0,), (0,)))
        d2 = m2 + c2_scr[...]
        mn = jnp.min(d2, axis=0, keepdims=True)
        iota = jax.lax.broadcasted_iota(jnp.int32, (K, B), 0).astype(jnp.float32)
        eq = d2 == mn
        labf = jnp.min(jnp.where(eq, iota, jnp.float32(K)),
                       axis=0, keepdims=True)
        lab_scr[b] = labf
        oh = eq.astype(jnp.bfloat16)
        sums_scr[...] += _dot(zbt, oh, ((1,), (1,)))
        cnt_scr[...] += _dot(jnp.ones((1, B), jnp.float32), oh, ((1,), (1,)))

        @pl.when(b == NB - 1)
        def _():
            cent_scr[...] = sums_scr[...] / jnp.maximum(cnt_scr[...], 1.0)

    @pl.when(jnp.logical_and(step >= NB, step < 2 * NB))
    def _phase_b():
        j = step - NB
        zbt = z_scr[j]
        labf = lab_scr[j]
        iota = jax.lax.broadcasted_iota(jnp.int32, (K, B), 0).astype(jnp.float32)
        oh = (iota == labf).astype(jnp.bfloat16)
        cdat = _dot(cent_scr[...], oh, ((1,), (0,)))
        diff = zbt - cdat
        dist = jnp.sum(diff * diff, axis=0, keepdims=True) * (1.0 / LAT)
        dist_scr[j] = dist

        @pl.when(j == 0)
        def _():
            mm_scr[0:1] = dist
            mm_scr[1:2] = dist

        @pl.when(j > 0)
        def _():
            mm_scr[0:1] = jnp.minimum(mm_scr[0:1], dist)
            mm_scr[1:2] = jnp.maximum(mm_scr[1:2], dist)

    @pl.when(step == 2 * NB)
    def _phase_c():
        mn = jnp.min(mm_scr[0:1], axis=1, keepdims=True)
        mx = jnp.max(mm_scr[1:2], axis=1, keepdims=True)

        def body(j, _):
            out_ref[j] = (dist_scr[j] - mn) / (mx - mn) + lab_scr[j]
            return 0

        jax.lax.fori_loop(0, NB, body, 0)


def kernel(table, W_e1, b_e1, W_e2, b_e2, W_mean, b_mean, W_logvar, b_logvar,
           W_d1, b_d1, W_d2, b_d2, BlockSize, current_epoch, baseline):
    out = pl.pallas_call(
        _fused_kernel,
        grid=(2 * NB + 1,),
        in_specs=[
            pl.BlockSpec((1, B, COL),
                         lambda s: (0, jnp.minimum(s, NB - 1), 0)),
            pl.BlockSpec((COL, 32), lambda s: (0, 0)),
            pl.BlockSpec((1, 32), lambda s: (0, 0)),
            pl.BlockSpec((32, 64), lambda s: (0, 0)),
            pl.BlockSpec((1, 64), lambda s: (0, 0)),
            pl.BlockSpec((LAT, 64), lambda s: (0, 0)),
            pl.BlockSpec((LAT, 1), lambda s: (0, 0)),
        ],
        out_specs=pl.BlockSpec((NB, 1, B), lambda s: (0, 0, 0)),
        out_shape=jax.ShapeDtypeStruct((NB, 1, B), jnp.float32),
        scratch_shapes=[
            pltpu.VMEM((NB, LAT, B), jnp.float32),
            pltpu.VMEM((NB, 1, B), jnp.float32),
            pltpu.VMEM((NB, 1, B), jnp.float32),
            pltpu.VMEM((LAT, K), jnp.float32),
            pltpu.VMEM((K, 1), jnp.float32),
            pltpu.VMEM((LAT, K), jnp.float32),
            pltpu.VMEM((LAT, K), jnp.float32),
            pltpu.VMEM((1, K), jnp.float32),
            pltpu.VMEM((2, B), jnp.float32),
        ],
    )(table, W_e1.T, b_e1.reshape(1, 32), W_e2.T, b_e2.reshape(1, 64),
      W_mean, b_mean.reshape(LAT, 1))
    return out.reshape(-1)


# speedup vs baseline: 1.2568x; 1.2568x over previous
import jax
import jax.numpy as jnp
from jax.experimental import pallas as pl
from jax.experimental.pallas import tpu as pltpu

ROWS = 100000
COL = 128
LAT = 16
K = 64
B = 20000
NB = ROWS // B


def _dot(a, b, dims):
    return jax.lax.dot_general(a, b, (dims, ((), ())),
                               preferred_element_type=jnp.float32)


def _fused_kernel(tab_ref, w1t_ref, b1_ref, w2t_ref, b2_ref, wm_ref, bm_ref,
                  out_ref,
                  z_scr, lab_scr, dist_scr, cent0_scr, c2_scr, cent_scr,
                  sums_scr, cnt_scr, mm_scr):
    step = pl.program_id(0)

    @pl.when(step < NB)
    def _phase_a():
        b = step
        x = tab_ref[0]
        h1 = jnp.maximum(_dot(x, w1t_ref[...], ((1,), (0,))) + b1_ref[...], 0.0)
        h2 = jnp.maximum(_dot(h1, w2t_ref[...], ((1,), (0,))) + b2_ref[...], 0.0)
        zbt = _dot(wm_ref[...], h2, ((1,), (1,))) + bm_ref[...]
        z_scr[b] = zbt

        @pl.when(b == 0)
        def _():
            c0 = zbt[:, :K]
            cent0_scr[...] = c0
            c2row = jnp.sum(c0 * c0, axis=0, keepdims=True)
            eye = (jax.lax.broadcasted_iota(jnp.int32, (K, K), 0)
                   == jax.lax.broadcasted_iota(jnp.int32, (K, K), 1))
            c2_scr[...] = jnp.sum(
                jnp.where(eye, jnp.broadcast_to(c2row, (K, K)), 0.0),
                axis=1, keepdims=True)
            sums_scr[...] = jnp.zeros_like(sums_scr)
            cnt_scr[...] = jnp.zeros_like(cnt_scr)

        m2 = _dot(-2.0 * cent0_scr[...], zbt, ((0,), (0,)))
        d2 = m2 + c2_scr[...]
        mn = jnp.min(d2, axis=0, keepdims=True)
        iota = jax.lax.broadcasted_iota(jnp.int32, (K, B), 0).astype(jnp.float32)
        eq = d2 == mn
        labf = jnp.min(jnp.where(eq, iota, jnp.float32(K)),
                       axis=0, keepdims=True)
        lab_scr[b] = labf
        oh = eq.astype(jnp.bfloat16)
        sums_scr[...] += _dot(zbt, oh, ((1,), (1,)))
        cnt_scr[...] += _dot(jnp.ones((1, B), jnp.float32), oh, ((1,), (1,)))

        @pl.when(b == NB - 1)
        def _():
            cent_scr[...] = sums_scr[...] / jnp.maximum(cnt_scr[...], 1.0)

    @pl.when(jnp.logical_and(step >= NB, step < 2 * NB))
    def _phase_b():
        j = step - NB
        zbt = z_scr[j]
        labf = lab_scr[j]
        iota = jax.lax.broadcasted_iota(jnp.int32, (K, B), 0).astype(jnp.float32)
        oh = (iota == labf).astype(jnp.bfloat16)
        cdat = _dot(cent_scr[...], oh, ((1,), (0,)))
        diff = zbt - cdat
        dist = jnp.sum(diff * diff, axis=0, keepdims=True) * (1.0 / LAT)
        dist_scr[j] = dist

        @pl.when(j == 0)
        def _():
            mm_scr[0:1] = dist
            mm_scr[1:2] = dist

        @pl.when(j > 0)
        def _():
            mm_scr[0:1] = jnp.minimum(mm_scr[0:1], dist)
            mm_scr[1:2] = jnp.maximum(mm_scr[1:2], dist)

    @pl.when(step == 2 * NB)
    def _phase_c():
        mn = jnp.min(mm_scr[0:1], axis=1, keepdims=True)
        mx = jnp.max(mm_scr[1:2], axis=1, keepdims=True)

        def body(j, _):
            out_ref[j] = (dist_scr[j] - mn) / (mx - mn) + lab_scr[j]
            return 0

        jax.lax.fori_loop(0, NB, body, 0)


def kernel(table, W_e1, b_e1, W_e2, b_e2, W_mean, b_mean, W_logvar, b_logvar,
           W_d1, b_d1, W_d2, b_d2, BlockSize, current_epoch, baseline):
    out = pl.pallas_call(
        _fused_kernel,
        grid=(2 * NB + 1,),
        in_specs=[
            pl.BlockSpec((1, B, COL),
                         lambda s: (0, jnp.minimum(s, NB - 1), 0)),
            pl.BlockSpec((COL, 32), lambda s: (0, 0)),
            pl.BlockSpec((1, 32), lambda s: (0, 0)),
            pl.BlockSpec((32, 64), lambda s: (0, 0)),
            pl.BlockSpec((1, 64), lambda s: (0, 0)),
            pl.BlockSpec((LAT, 64), lambda s: (0, 0)),
            pl.BlockSpec((LAT, 1), lambda s: (0, 0)),
        ],
        out_specs=pl.BlockSpec((NB, 1, B), lambda s: (0, 0, 0)),
        out_shape=jax.ShapeDtypeStruct((NB, 1, B), jnp.float32),
        scratch_shapes=[
            pltpu.VMEM((NB, LAT, B), jnp.float32),
            pltpu.VMEM((NB, 1, B), jnp.float32),
            pltpu.VMEM((NB, 1, B), jnp.float32),
            pltpu.VMEM((LAT, K), jnp.float32),
            pltpu.VMEM((K, 1), jnp.float32),
            pltpu.VMEM((LAT, K), jnp.float32),
            pltpu.VMEM((LAT, K), jnp.float32),
            pltpu.VMEM((1, K), jnp.float32),
            pltpu.VMEM((2, B), jnp.float32),
        ],
    )(table, W_e1.T, b_e1.reshape(1, 32), W_e2.T, b_e2.reshape(1, 64),
      W_mean, b_mean.reshape(LAT, 1))
    return out.reshape(-1)
